# baseline (speedup 1.0000x reference)
import functools

import jax
import jax.numpy as jnp
import numpy as np
from jax import lax
from jax.experimental import pallas as pl
from jax.experimental.pallas import tpu as pltpu
from jax.experimental.pallas import tpu_sc as plsc

B = 16384
D = 128
C = 1000
C_PAD = 1024
K_OHEM = int(B * 0.7)
TB = 2048
NT = B // TB

NW = 32
EPW = B // NW
ECH = 128
TAB_W = D + 16

_MSB = np.uint32(0x80000000)
_U1 = np.uint32(1)
_U31 = np.uint32(31)


def _tdot_body(tab_ref, tgt_ref, f_ref, out_ref, idx_v, rows_v, feat_v, out_v, sem):
    wid = lax.axis_index("s") * 2 + lax.axis_index("c")
    base = wid * EPW
    lane = lax.iota(jnp.int32, 16)

    def chunk(c, carry):
        cb = base + c * ECH
        pltpu.sync_copy(tgt_ref.at[pl.ds(cb, ECH)], idx_v)
        pltpu.async_copy(tab_ref.at[idx_v], rows_v, sem).wait()
        pltpu.sync_copy(f_ref.at[pl.ds(cb, ECH)], feat_v)

        def grp(g, carry2):
            e16 = g * 16 + lane

            def dstep(d, acc):
                dcol = jnp.full((16,), d, jnp.int32)
                w = plsc.load_gather(rows_v, [e16, dcol])
                fv = plsc.load_gather(feat_v, [e16, dcol])
                return acc + w * fv

            acc0 = plsc.load_gather(rows_v, [e16, jnp.full((16,), D, jnp.int32)])
            acc = lax.fori_loop(0, D, dstep, acc0)
            out_v[pl.ds(c * ECH + g * 16, 16)] = acc
            return carry2

        return lax.fori_loop(0, ECH // 16, grp, carry)

    lax.fori_loop(0, EPW // ECH, chunk, 0)
    pltpu.sync_copy(out_v, out_ref.at[pl.ds(base, EPW)])


def _tdot_call(tab, tgt, features):
    return pl.kernel(
        _tdot_body,
        out_type=jax.ShapeDtypeStruct((B,), jnp.float32),
        mesh=plsc.VectorSubcoreMesh(core_axis_name="c", subcore_axis_name="s"),
        scratch_types=[
            pltpu.VMEM((ECH,), jnp.int32),
            pltpu.VMEM((ECH, TAB_W), jnp.float32),
            pltpu.VMEM((ECH, D), jnp.float32),
            pltpu.VMEM((EPW,), jnp.float32),
            pltpu.SemaphoreType.DMA,
        ],
        compiler_params=pltpu.CompilerParams(
            use_tc_tiling_on_sc=False, needs_layout_passes=False
        ),
    )(tab, tgt, features)


def _tree_reduce(v, combine, final):
    n = v.shape[0]
    while n > 16:
        n //= 2
        v = combine(v[:n], v[n:])
    return final(v.astype(jnp.float32), axis=0, keepdims=True)


def _lse_kernel(wt_ref, f_ref, b_ref, out_ref):
    fb = f_ref[...].astype(jnp.bfloat16)
    x32 = jax.lax.dot_general(
        wt_ref[...], fb, (((1,), (1,)), ((), ())),
        preferred_element_type=jnp.float32,
    )
    x = x32.astype(jnp.bfloat16) + b_ref[...]
    m = _tree_reduce(x, jnp.maximum, jnp.max)
    e = jnp.exp(x - m.astype(jnp.bfloat16))
    s = _tree_reduce(e, jnp.add, jnp.sum)
    out_ref[...] = (m + jnp.log(s))[None]


def _select_kernel(lse_ref, td_ref, out_ref):
    losses = lse_ref[...] - td_ref[...]
    u = jax.lax.bitcast_convert_type(losses, jnp.uint32)
    key = jnp.where((u & _MSB) != 0, ~u, u | _MSB)

    def body(j, p):
        cand = p | (_U1 << (_U31 - j.astype(jnp.uint32)))
        cnt = jnp.sum((key >= cand).astype(jnp.int32))
        return jnp.where(cnt >= K_OHEM, cand, p)

    t_key = jax.lax.fori_loop(0, 16, body, jnp.zeros((), jnp.uint32))
    t_bits = jnp.where((t_key & _MSB) != 0, t_key ^ _MSB, ~t_key)
    t_val = jax.lax.bitcast_convert_type(t_bits, jnp.float32)
    gt = key > t_key
    cnt_gt = jnp.sum(gt.astype(jnp.int32))
    sum_gt = jnp.sum(jnp.where(gt, losses, 0.0))
    mean = (sum_gt + (K_OHEM - cnt_gt).astype(jnp.float32) * t_val) / K_OHEM
    out_ref[...] = jnp.reshape(mean, (1, 1))


@jax.jit
def kernel(features, targets, W, b):
    wt = jnp.zeros((C_PAD, D), jnp.bfloat16).at[:C, :].set(W.T.astype(jnp.bfloat16))
    bias = jnp.broadcast_to(
        jnp.concatenate(
            [b.astype(jnp.bfloat16), jnp.full((C_PAD - C,), -1e30, jnp.bfloat16)]
        )[:, None],
        (C_PAD, TB),
    )
    tab = (
        jnp.zeros((C_PAD, TAB_W), jnp.float32)
        .at[:C, :D].set(W.T)
        .at[:C, D].set(b)
    )
    tgt = targets.astype(jnp.int32)

    tdot = _tdot_call(tab, tgt, features)

    lse = pl.pallas_call(
        _lse_kernel,
        grid=(NT,),
        in_specs=[
            pl.BlockSpec((C_PAD, D), lambda i: (0, 0)),
            pl.BlockSpec((TB, D), lambda i: (i, 0)),
            pl.BlockSpec((C_PAD, TB), lambda i: (0, 0)),
        ],
        out_specs=pl.BlockSpec((1, 1, TB), lambda i: (i, 0, 0)),
        out_shape=jax.ShapeDtypeStruct((NT, 1, TB), jnp.float32),
        compiler_params=pltpu.CompilerParams(
            dimension_semantics=("arbitrary",),
        ),
    )(wt, features, bias)

    out = pl.pallas_call(
        _select_kernel,
        in_specs=[
            pl.BlockSpec((NT, 1, TB), lambda: (0, 0, 0)),
            pl.BlockSpec((NT, 1, TB), lambda: (0, 0, 0)),
        ],
        out_specs=pl.BlockSpec((1, 1), lambda: (0, 0)),
        out_shape=jax.ShapeDtypeStruct((1, 1), jnp.float32),
    )(lse, tdot.reshape(NT, 1, TB))
    return out.reshape(())


# speedup vs baseline: 1.0064x; 1.0064x over previous
import functools

import jax
import jax.numpy as jnp
import numpy as np
from jax import lax
from jax.experimental import pallas as pl
from jax.experimental.pallas import tpu as pltpu
from jax.experimental.pallas import tpu_sc as plsc

B = 16384
D = 128
C = 1000
C_PAD = 1024
K_OHEM = int(B * 0.7)
TB = 2048
NT = B // TB

NW = 32
EPW = B // NW
ECH = 128
TAB_W = D + 16

_MSB = np.uint32(0x80000000)
_U1 = np.uint32(1)
_U31 = np.uint32(31)


def _tdot_body(tab_ref, tgt_ref, f_ref, out_ref, idx_v, rows_v, feat_v, out_v, sem):
    wid = lax.axis_index("s") * 2 + lax.axis_index("c")
    base = wid * EPW
    lane = lax.iota(jnp.int32, 16)

    def chunk(c, carry):
        cb = base + c * ECH
        pltpu.sync_copy(tgt_ref.at[pl.ds(cb, ECH)], idx_v)
        pltpu.async_copy(tab_ref.at[idx_v], rows_v, sem).wait()
        pltpu.sync_copy(f_ref.at[pl.ds(cb, ECH)], feat_v)

        def grp(g, carry2):
            e16 = g * 16 + lane
            acc = plsc.load_gather(rows_v, [e16, jnp.full((16,), D, jnp.int32)])
            for d in range(D):
                dcol = jnp.full((16,), d, jnp.int32)
                w = plsc.load_gather(rows_v, [e16, dcol])
                fv = plsc.load_gather(feat_v, [e16, dcol])
                acc = acc + w * fv
            out_v[pl.ds(c * ECH + g * 16, 16)] = acc
            return carry2

        return lax.fori_loop(0, ECH // 16, grp, carry)

    lax.fori_loop(0, EPW // ECH, chunk, 0)
    pltpu.sync_copy(out_v, out_ref.at[pl.ds(base, EPW)])


def _tdot_call(tab, tgt, features):
    return pl.kernel(
        _tdot_body,
        out_type=jax.ShapeDtypeStruct((B,), jnp.float32),
        mesh=plsc.VectorSubcoreMesh(core_axis_name="c", subcore_axis_name="s"),
        scratch_types=[
            pltpu.VMEM((ECH,), jnp.int32),
            pltpu.VMEM((ECH, TAB_W), jnp.float32),
            pltpu.VMEM((ECH, D), jnp.float32),
            pltpu.VMEM((EPW,), jnp.float32),
            pltpu.SemaphoreType.DMA,
        ],
        compiler_params=pltpu.CompilerParams(
            use_tc_tiling_on_sc=False, needs_layout_passes=False
        ),
    )(tab, tgt, features)


def _tree_reduce(v, combine, final):
    n = v.shape[0]
    while n > 16:
        n //= 2
        v = combine(v[:n], v[n:])
    return final(v.astype(jnp.float32), axis=0, keepdims=True)


def _lse_kernel(wt_ref, f_ref, b_ref, out_ref):
    fb = f_ref[...].astype(jnp.bfloat16)
    x32 = jax.lax.dot_general(
        wt_ref[...], fb, (((1,), (1,)), ((), ())),
        preferred_element_type=jnp.float32,
    )
    x = x32.astype(jnp.bfloat16) + b_ref[...]
    m = _tree_reduce(x, jnp.maximum, jnp.max)
    e = jnp.exp(x - m.astype(jnp.bfloat16))
    s = _tree_reduce(e, jnp.add, jnp.sum)
    out_ref[...] = (m + jnp.log(s))[None]


def _select_kernel(lse_ref, td_ref, out_ref):
    losses = lse_ref[...] - td_ref[...]
    u = jax.lax.bitcast_convert_type(losses, jnp.uint32)
    key = jnp.where((u & _MSB) != 0, ~u, u | _MSB)

    def body(j, p):
        cand = p | (_U1 << (_U31 - j.astype(jnp.uint32)))
        cnt = jnp.sum((key >= cand).astype(jnp.int32))
        return jnp.where(cnt >= K_OHEM, cand, p)

    t_key = jax.lax.fori_loop(0, 16, body, jnp.zeros((), jnp.uint32))
    t_bits = jnp.where((t_key & _MSB) != 0, t_key ^ _MSB, ~t_key)
    t_val = jax.lax.bitcast_convert_type(t_bits, jnp.float32)
    gt = key > t_key
    cnt_gt = jnp.sum(gt.astype(jnp.int32))
    sum_gt = jnp.sum(jnp.where(gt, losses, 0.0))
    mean = (sum_gt + (K_OHEM - cnt_gt).astype(jnp.float32) * t_val) / K_OHEM
    out_ref[...] = jnp.reshape(mean, (1, 1))


@jax.jit
def kernel(features, targets, W, b):
    wt = jnp.zeros((C_PAD, D), jnp.bfloat16).at[:C, :].set(W.T.astype(jnp.bfloat16))
    bias = jnp.broadcast_to(
        jnp.concatenate(
            [b.astype(jnp.bfloat16), jnp.full((C_PAD - C,), -1e30, jnp.bfloat16)]
        )[:, None],
        (C_PAD, TB),
    )
    tab = (
        jnp.zeros((C_PAD, TAB_W), jnp.float32)
        .at[:C, :D].set(W.T)
        .at[:C, D].set(b)
    )
    tgt = targets.astype(jnp.int32)

    tdot = _tdot_call(tab, tgt, features)

    lse = pl.pallas_call(
        _lse_kernel,
        grid=(NT,),
        in_specs=[
            pl.BlockSpec((C_PAD, D), lambda i: (0, 0)),
            pl.BlockSpec((TB, D), lambda i: (i, 0)),
            pl.BlockSpec((C_PAD, TB), lambda i: (0, 0)),
        ],
        out_specs=pl.BlockSpec((1, 1, TB), lambda i: (i, 0, 0)),
        out_shape=jax.ShapeDtypeStruct((NT, 1, TB), jnp.float32),
        compiler_params=pltpu.CompilerParams(
            dimension_semantics=("arbitrary",),
        ),
    )(wt, features, bias)

    out = pl.pallas_call(
        _select_kernel,
        in_specs=[
            pl.BlockSpec((NT, 1, TB), lambda: (0, 0, 0)),
            pl.BlockSpec((NT, 1, TB), lambda: (0, 0, 0)),
        ],
        out_specs=pl.BlockSpec((1, 1), lambda: (0, 0)),
        out_shape=jax.ShapeDtypeStruct((1, 1), jnp.float32),
    )(lse, tdot.reshape(NT, 1, TB))
    return out.reshape(())


# speedup vs baseline: 1.7271x; 1.7162x over previous
import jax
import jax.numpy as jnp
import numpy as np
from jax.experimental import pallas as pl
from jax.experimental.pallas import tpu as pltpu

B = 16384
D = 128
C = 1000
C_PAD = 1024
K_OHEM = int(B * 0.7)
TB = 2048
NT = B // TB

_MSB = np.uint32(0x80000000)
_U1 = np.uint32(1)
_U31 = np.uint32(31)


def _tree_reduce(v, combine, final):
    n = v.shape[0]
    while n > 16:
        n //= 2
        v = combine(v[:n], v[n:])
    return final(v.astype(jnp.float32), axis=0, keepdims=True)


def _ohem_kernel(wt_ref, f_ref, b_ref, tgt_ref, out_ref, loss_scratch):
    i = pl.program_id(0)
    fb = f_ref[...].astype(jnp.bfloat16)
    x32 = jax.lax.dot_general(
        wt_ref[...], fb, (((1,), (1,)), ((), ())),
        preferred_element_type=jnp.float32,
    )
    bb = b_ref[...]
    x = (
        x32.reshape(C_PAD, TB // 128, 128).astype(jnp.bfloat16) + bb[:, None, :]
    ).reshape(C_PAD, TB)
    m = _tree_reduce(x, jnp.maximum, jnp.max)
    e = jnp.exp(x - m.astype(jnp.bfloat16))
    s = _tree_reduce(e, jnp.add, jnp.sum)
    lse = m + jnp.log(s)
    tgt = tgt_ref[0].astype(jnp.int16)
    rows = jax.lax.broadcasted_iota(jnp.int16, (C_PAD, TB), 0)
    tlogit = _tree_reduce(
        jnp.where(rows == tgt, x, jnp.bfloat16(0.0)), jnp.add, jnp.sum
    )
    loss_scratch[pl.ds(i, 1), :] = lse - tlogit

    @pl.when(i == NT - 1)
    def _select():
        losses = loss_scratch[...]
        u = jax.lax.bitcast_convert_type(losses, jnp.uint32)
        key = jnp.where((u & _MSB) != 0, ~u, u | _MSB)

        def body(j, p):
            cand = p | (_U1 << (_U31 - j.astype(jnp.uint32)))
            cnt = jnp.sum((key >= cand).astype(jnp.int32))
            return jnp.where(cnt >= K_OHEM, cand, p)

        t_key = jax.lax.fori_loop(0, 16, body, jnp.zeros((), jnp.uint32))
        t_bits = jnp.where((t_key & _MSB) != 0, t_key ^ _MSB, ~t_key)
        t_val = jax.lax.bitcast_convert_type(t_bits, jnp.float32)
        gt = key > t_key
        cnt_gt = jnp.sum(gt.astype(jnp.int32))
        sum_gt = jnp.sum(jnp.where(gt, losses, 0.0))
        mean = (sum_gt + (K_OHEM - cnt_gt).astype(jnp.float32) * t_val) / K_OHEM
        out_ref[...] = jnp.reshape(mean, (1, 1))


@jax.jit
def kernel(features, targets, W, b):
    wt = jnp.zeros((C_PAD, D), jnp.bfloat16).at[:C, :].set(W.T.astype(jnp.bfloat16))
    bias = jnp.broadcast_to(
        jnp.concatenate(
            [b.astype(jnp.bfloat16), jnp.full((C_PAD - C,), -1e30, jnp.bfloat16)]
        )[:, None],
        (C_PAD, 128),
    )
    tgt = targets.astype(jnp.int32).reshape(NT, 1, TB)

    out = pl.pallas_call(
        _ohem_kernel,
        grid=(NT,),
        in_specs=[
            pl.BlockSpec((C_PAD, D), lambda i: (0, 0)),
            pl.BlockSpec((TB, D), lambda i: (i, 0)),
            pl.BlockSpec((C_PAD, 128), lambda i: (0, 0)),
            pl.BlockSpec((1, 1, TB), lambda i: (i, 0, 0)),
        ],
        out_specs=pl.BlockSpec((1, 1), lambda i: (0, 0)),
        out_shape=jax.ShapeDtypeStruct((1, 1), jnp.float32),
        scratch_shapes=[pltpu.VMEM((NT, TB), jnp.float32)],
        compiler_params=pltpu.CompilerParams(
            dimension_semantics=("arbitrary",),
        ),
    )(wt, features, bias, tgt)
    return out.reshape(())


# speedup vs baseline: 2.6518x; 1.5354x over previous
import jax
import jax.numpy as jnp
import numpy as np
from jax.experimental import pallas as pl
from jax.experimental.pallas import tpu as pltpu

B = 16384
D = 128
C = 1000
C_PAD = 1024
K_OHEM = int(B * 0.7)
TB = 2048
NT = B // TB

_MSB = np.uint32(0x80000000)
_U1 = np.uint32(1)
_U31 = np.uint32(31)


def _tree_reduce(v, combine, final):
    n = v.shape[0]
    while n > 16:
        n //= 2
        v = combine(v[:n], v[n:])
    return final(v.astype(jnp.float32), axis=0, keepdims=True)


def _ohem_kernel(wt_ref, f_ref, b_ref, tgt_ref, out_ref, loss_scratch):
    i = pl.program_id(0)
    fb = f_ref[...].astype(jnp.bfloat16)
    x32 = jax.lax.dot_general(
        wt_ref[...], fb, (((1,), (1,)), ((), ())),
        preferred_element_type=jnp.float32,
    )
    x = x32.astype(jnp.bfloat16) + b_ref[...]
    m = _tree_reduce(x, jnp.maximum, jnp.max)
    e = jnp.exp(x - m.astype(jnp.bfloat16))
    s8 = jax.lax.dot_general(
        jnp.ones((8, C_PAD), jnp.bfloat16), e, (((1,), (0,)), ((), ())),
        preferred_element_type=jnp.float32,
    )
    lse = m + jnp.log(s8[:1])
    tgt = tgt_ref[0].astype(jnp.int16)
    rows = jax.lax.broadcasted_iota(jnp.int16, (C_PAD, TB), 0)
    tlogit = _tree_reduce(
        jnp.where(rows == tgt, x, jnp.bfloat16(0.0)), jnp.add, jnp.sum
    )
    loss_scratch[pl.ds(i, 1), :] = lse - tlogit

    @pl.when(i == NT - 1)
    def _select():
        losses = loss_scratch[...]
        u = jax.lax.bitcast_convert_type(losses, jnp.uint32)
        key = jnp.where((u & _MSB) != 0, ~u, u | _MSB)

        def body(j, p):
            cand = p | (_U1 << (_U31 - j.astype(jnp.uint32)))
            cnt = jnp.sum((key >= cand).astype(jnp.int32))
            return jnp.where(cnt >= K_OHEM, cand, p)

        t_key = jax.lax.fori_loop(0, 16, body, jnp.zeros((), jnp.uint32))
        t_bits = jnp.where((t_key & _MSB) != 0, t_key ^ _MSB, ~t_key)
        t_val = jax.lax.bitcast_convert_type(t_bits, jnp.float32)
        gt = key > t_key
        cnt_gt = jnp.sum(gt.astype(jnp.int32))
        sum_gt = jnp.sum(jnp.where(gt, losses, 0.0))
        mean = (sum_gt + (K_OHEM - cnt_gt).astype(jnp.float32) * t_val) / K_OHEM
        out_ref[...] = jnp.reshape(mean, (1, 1))


@jax.jit
def kernel(features, targets, W, b):
    wt = jnp.zeros((C_PAD, D), jnp.bfloat16).at[:C, :].set(W.T.astype(jnp.bfloat16))
    bias = jnp.broadcast_to(
        jnp.concatenate(
            [b.astype(jnp.bfloat16), jnp.full((C_PAD - C,), -1e30, jnp.bfloat16)]
        )[:, None],
        (C_PAD, TB),
    )
    tgt = targets.astype(jnp.int32).reshape(NT, 1, TB)

    out = pl.pallas_call(
        _ohem_kernel,
        grid=(NT,),
        in_specs=[
            pl.BlockSpec((C_PAD, D), lambda i: (0, 0)),
            pl.BlockSpec((TB, D), lambda i: (i, 0)),
            pl.BlockSpec((C_PAD, TB), lambda i: (0, 0)),
            pl.BlockSpec((1, 1, TB), lambda i: (i, 0, 0)),
        ],
        out_specs=pl.BlockSpec((1, 1), lambda i: (0, 0)),
        out_shape=jax.ShapeDtypeStruct((1, 1), jnp.float32),
        scratch_shapes=[pltpu.VMEM((NT, TB), jnp.float32)],
        compiler_params=pltpu.CompilerParams(
            dimension_semantics=("arbitrary",),
        ),
    )(wt, features, bias, tgt)
    return out.reshape(())


# speedup vs baseline: 2.8777x; 1.0852x over previous
import jax
import jax.numpy as jnp
import numpy as np
from jax.experimental import pallas as pl
from jax.experimental.pallas import tpu as pltpu

B = 16384
D = 128
C = 1000
C_PAD = 1024
K_OHEM = int(B * 0.7)
TB = 2048
NT = B // TB

_MSB = np.uint32(0x80000000)
_U1 = np.uint32(1)
_U31 = np.uint32(31)


def _tree_reduce(v, combine, final):
    n = v.shape[0]
    while n > 16:
        n //= 4
        v = combine(
            combine(v[:n], v[n : 2 * n]),
            combine(v[2 * n : 3 * n], v[3 * n :]),
        )
    return final(v.astype(jnp.float32), axis=0, keepdims=True)


def _ohem_kernel(wt_ref, f_ref, b_ref, tgt_ref, out_ref, loss_scratch):
    i = pl.program_id(0)
    fb = f_ref[...].astype(jnp.bfloat16)
    x32 = jax.lax.dot_general(
        wt_ref[...], fb, (((1,), (1,)), ((), ())),
        preferred_element_type=jnp.float32,
    )
    x = x32.astype(jnp.bfloat16) + b_ref[...]
    m = _tree_reduce(x, jnp.maximum, jnp.max)
    e = jnp.exp(x - m.astype(jnp.bfloat16))
    s = _tree_reduce(e, jnp.add, jnp.sum)
    lse = m + jnp.log(s)
    tgt = tgt_ref[0].astype(jnp.int16)
    rows = jax.lax.broadcasted_iota(jnp.int16, (C_PAD, TB), 0)
    tlogit = _tree_reduce(
        jnp.where(rows == tgt, x, jnp.bfloat16(0.0)), jnp.add, jnp.sum
    )
    loss_scratch[pl.ds(i, 1), :] = lse - tlogit

    @pl.when(i == NT - 1)
    def _select():
        losses = loss_scratch[...]
        u = jax.lax.bitcast_convert_type(losses, jnp.uint32)
        key = jnp.where((u & _MSB) != 0, ~u, u | _MSB)

        def body(j, p):
            cand = p | (_U1 << (_U31 - j.astype(jnp.uint32)))
            cnt = jnp.sum((key >= cand).astype(jnp.int32))
            return jnp.where(cnt >= K_OHEM, cand, p)

        t_key = jax.lax.fori_loop(0, 16, body, jnp.zeros((), jnp.uint32))
        t_bits = jnp.where((t_key & _MSB) != 0, t_key ^ _MSB, ~t_key)
        t_val = jax.lax.bitcast_convert_type(t_bits, jnp.float32)
        gt = key > t_key
        cnt_gt = jnp.sum(gt.astype(jnp.int32))
        sum_gt = jnp.sum(jnp.where(gt, losses, 0.0))
        mean = (sum_gt + (K_OHEM - cnt_gt).astype(jnp.float32) * t_val) / K_OHEM
        out_ref[...] = jnp.reshape(mean, (1, 1))


@jax.jit
def kernel(features, targets, W, b):
    wt = jnp.zeros((C_PAD, D), jnp.bfloat16).at[:C, :].set(W.T.astype(jnp.bfloat16))
    bias = jnp.broadcast_to(
        jnp.concatenate(
            [b.astype(jnp.bfloat16), jnp.full((C_PAD - C,), -1e30, jnp.bfloat16)]
        )[:, None],
        (C_PAD, TB),
    )
    tgt = targets.astype(jnp.int32).reshape(NT, 1, TB)

    out = pl.pallas_call(
        _ohem_kernel,
        grid=(NT,),
        in_specs=[
            pl.BlockSpec((C_PAD, D), lambda i: (0, 0)),
            pl.BlockSpec((TB, D), lambda i: (i, 0)),
            pl.BlockSpec((C_PAD, TB), lambda i: (0, 0)),
            pl.BlockSpec((1, 1, TB), lambda i: (i, 0, 0)),
        ],
        out_specs=pl.BlockSpec((1, 1), lambda i: (0, 0)),
        out_shape=jax.ShapeDtypeStruct((1, 1), jnp.float32),
        scratch_shapes=[pltpu.VMEM((NT, TB), jnp.float32)],
        compiler_params=pltpu.CompilerParams(
            dimension_semantics=("arbitrary",),
        ),
    )(wt, features, bias, tgt)
    return out.reshape(())


# speedup vs baseline: 3.0241x; 1.0509x over previous
import jax
import jax.numpy as jnp
import numpy as np
from jax.experimental import pallas as pl
from jax.experimental.pallas import tpu as pltpu

B = 16384
D = 128
C = 1000
C_PAD = 1024
K_OHEM = int(B * 0.7)
TB = 2048
NT = B // TB

_MSB = np.uint32(0x80000000)
_U1 = np.uint32(1)
_U31 = np.uint32(31)


def _tree_reduce(v, combine, final):
    n = v.shape[0]
    while n > 16:
        n //= 4
        v = combine(
            combine(v[:n], v[n : 2 * n]),
            combine(v[2 * n : 3 * n], v[3 * n :]),
        )
    return final(v.astype(jnp.float32), axis=0, keepdims=True)


def _ohem_kernel(wt_ref, f_ref, b_ref, tgt_ref, out_ref, loss_scratch, bias_sc):
    i = pl.program_id(0)

    @pl.when(i == 0)
    def _bcast_bias():
        bias_sc[...] = jnp.concatenate([b_ref[...]] * (TB // 128), axis=1)

    fb = f_ref[...].astype(jnp.bfloat16)
    x32 = jax.lax.dot_general(
        wt_ref[...], fb, (((1,), (1,)), ((), ())),
        preferred_element_type=jnp.float32,
    )
    x = x32.astype(jnp.bfloat16) + bias_sc[...]
    m = _tree_reduce(x, jnp.maximum, jnp.max)
    e = jnp.exp(x - m.astype(jnp.bfloat16))
    s = _tree_reduce(e, jnp.add, jnp.sum)
    lse = m + jnp.log(s)
    tgt = tgt_ref[0].astype(jnp.int16)
    rows = jax.lax.broadcasted_iota(jnp.int16, (C_PAD, TB), 0)
    tlogit = _tree_reduce(
        jnp.where(rows == tgt, x, jnp.bfloat16(0.0)), jnp.add, jnp.sum
    )
    loss_scratch[pl.ds(i, 1), :] = lse - tlogit

    @pl.when(i == NT - 1)
    def _select():
        losses = loss_scratch[...]
        u = jax.lax.bitcast_convert_type(losses, jnp.uint32)
        key = jnp.where((u & _MSB) != 0, ~u, u | _MSB)

        def body(j, p):
            cand = p | (_U1 << (_U31 - j.astype(jnp.uint32)))
            cnt = jnp.sum((key >= cand).astype(jnp.int32))
            return jnp.where(cnt >= K_OHEM, cand, p)

        t_key = jax.lax.fori_loop(0, 16, body, jnp.zeros((), jnp.uint32))
        t_bits = jnp.where((t_key & _MSB) != 0, t_key ^ _MSB, ~t_key)
        t_val = jax.lax.bitcast_convert_type(t_bits, jnp.float32)
        gt = key > t_key
        cnt_gt = jnp.sum(gt.astype(jnp.int32))
        sum_gt = jnp.sum(jnp.where(gt, losses, 0.0))
        mean = (sum_gt + (K_OHEM - cnt_gt).astype(jnp.float32) * t_val) / K_OHEM
        out_ref[...] = jnp.reshape(mean, (1, 1))


@jax.jit
def kernel(features, targets, W, b):
    wt = jnp.zeros((C_PAD, D), jnp.bfloat16).at[:C, :].set(W.T.astype(jnp.bfloat16))
    bias = jnp.broadcast_to(
        jnp.concatenate(
            [b.astype(jnp.bfloat16), jnp.full((C_PAD - C,), -1e30, jnp.bfloat16)]
        )[:, None],
        (C_PAD, 128),
    )
    tgt = targets.astype(jnp.int32).reshape(NT, 1, TB)

    out = pl.pallas_call(
        _ohem_kernel,
        grid=(NT,),
        in_specs=[
            pl.BlockSpec((C_PAD, D), lambda i: (0, 0)),
            pl.BlockSpec((TB, D), lambda i: (i, 0)),
            pl.BlockSpec((C_PAD, 128), lambda i: (0, 0)),
            pl.BlockSpec((1, 1, TB), lambda i: (i, 0, 0)),
        ],
        out_specs=pl.BlockSpec((1, 1), lambda i: (0, 0)),
        out_shape=jax.ShapeDtypeStruct((1, 1), jnp.float32),
        scratch_shapes=[
            pltpu.VMEM((NT, TB), jnp.float32),
            pltpu.VMEM((C_PAD, TB), jnp.bfloat16),
        ],
        compiler_params=pltpu.CompilerParams(
            dimension_semantics=("arbitrary",),
        ),
    )(wt, features, bias, tgt)
    return out.reshape(())
